# baseline (speedup 1.0000x reference)
import functools

import jax
import jax.numpy as jnp
from jax import lax
from jax.experimental import pallas as pl
from jax.experimental.pallas import tpu as pltpu
from jax.experimental.pallas import tpu_sc as plsc


def kernel(user, item, user_table, item_table):
    B = user.shape[0]
    D = user_table.shape[1]

    info = plsc.get_sparse_core_info()
    NC, NS, L = info.num_cores, info.num_subcores, info.num_lanes
    NW = NC * NS
    bpw = B // NW

    @functools.partial(
        pl.kernel,
        out_type=jax.ShapeDtypeStruct((B,), jnp.float32),
        mesh=plsc.VectorSubcoreMesh(core_axis_name="c", subcore_axis_name="s"),
        compiler_params=pltpu.CompilerParams(
            use_tc_tiling_on_sc=False, needs_layout_passes=False
        ),
        scratch_types=[
            pltpu.VMEM((bpw,), jnp.int32),
            pltpu.VMEM((bpw,), jnp.int32),
            pltpu.VMEM((bpw, D), jnp.float32),
            pltpu.VMEM((bpw, D), jnp.float32),
            pltpu.VMEM((bpw,), jnp.float32),
            pltpu.SemaphoreType.DMA,
        ],
    )
    def mf_kernel(user_hbm, item_hbm, ut_hbm, it_hbm, out_hbm,
                  uidx_v, iidx_v, urows_v, irows_v, out_v, sem):
        wid = lax.axis_index("s") * NC + lax.axis_index("c")
        base = wid * bpw

        pltpu.sync_copy(user_hbm.at[pl.ds(base, bpw)], uidx_v)
        pltpu.sync_copy(item_hbm.at[pl.ds(base, bpw)], iidx_v)

        cu = pltpu.async_copy(ut_hbm.at[uidx_v], urows_v, sem)
        ci = pltpu.async_copy(it_hbm.at[iidx_v], irows_v, sem)
        cu.wait()
        ci.wait()

        lanes = lax.iota(jnp.int32, L)

        def body(g, carry):
            row = g * L + lanes
            acc = jnp.zeros((L,), jnp.float32)
            for d in range(D):
                col = jnp.full((L,), d, jnp.int32)
                uu = plsc.load_gather(urows_v, [row, col])
                ii = plsc.load_gather(irows_v, [row, col])
                acc = acc + uu * ii
            out_v[pl.ds(g * L, L)] = acc
            return carry

        lax.fori_loop(0, bpw // L, body, 0)

        pltpu.sync_copy(out_v, out_hbm.at[pl.ds(base, bpw)])

    return mf_kernel(user, item, user_table, item_table)
